# Initial kernel scaffold; baseline (speedup 1.0000x reference)
import jax
import jax.numpy as jnp
from jax.experimental import pallas as pl


def kernel(R, Z, N, chg_embed, chg_Wf1, chg_bf1, chg_Wf2, chg_bf2, chg_Wu, chg_bu, chg_Wa, chg_Wp, dlt_embed, dlt_Wf1, dlt_bf1, dlt_Wf2, dlt_bf2, dlt_Wu, dlt_bu, dlt_Wa, dlt_Wp):
    raise NotImplementedError("write your pallas kernel here")


# speedup vs baseline: 11.1482x; 11.1482x over previous
import jax
import jax.numpy as jnp
import numpy as np
from jax.experimental import pallas as pl
from jax.experimental.pallas import tpu as pltpu

B, A, NN, F, RES, T = 16, 128, 32, 128, 20, 3
E = A * NN
CUTOFF = 5.0
K = 332.063711
NZ = 100


def _centers():
    k = jax.lax.broadcasted_iota(jnp.int32, (1, RES), 1).astype(jnp.float32)
    return k * (CUTOFF / (RES - 1))


def _dot(a, b):
    return jax.lax.dot_general(a, b, (((1,), (0,)), ((), ())),
                               preferred_element_type=jnp.float32)


def _dotT(a, b):
    return jax.lax.dot_general(a, b, (((0,), (0,)), ((), ())),
                               preferred_element_type=jnp.float32)


def _dotBT(a, b):
    return jax.lax.dot_general(a, b, (((1,), (1,)), ((), ())),
                               preferred_element_type=jnp.float32)


def _dotTH(a, b):
    return jax.lax.dot_general(a, b, (((0,), (0,)), ((), ())),
                               precision=jax.lax.Precision.HIGHEST,
                               preferred_element_type=jnp.float32)


def _silu(x):
    return x * jax.nn.sigmoid(x)


def _dsilu(x):
    s = jax.nn.sigmoid(x)
    return s * (1.0 + x * (1.0 - s))


def _fwd(S, rbf, fcE, h0, Wf1_ref, bf1_ref, Wf2_ref, bf2_ref, Wu_ref, bu_ref,
         hs_ref):
    hs_ref[0] = h0

    def step(t, h):
        s1 = _dot(rbf, Wf1_ref[t]) + bf1_ref[t]
        w = _dot(_silu(s1), Wf2_ref[t]) + bf2_ref[t]
        hj = _dotT(S, h)
        m = jnp.sum((hj * w * fcE).reshape(A, NN, F), axis=1)
        h = h + _silu(_dot(m, Wu_ref[t]) + bu_ref[t])
        hs_ref[pl.ds(t + 1, 1)] = h[None]
        return h

    return jax.lax.fori_loop(0, T, step, h0)


def _bwd(S, rbf, fcE, hs_ref, g_h3, Wf1_ref, bf1_ref, Wf2_ref, bf2_ref,
         Wu_ref, bu_ref):
    def step(r, carry):
        g_h, g_rbf, g_fc = carry
        t = T - 1 - r
        hprev = hs_ref[t]
        s1 = _dot(rbf, Wf1_ref[t]) + bf1_ref[t]
        w = _dot(_silu(s1), Wf2_ref[t]) + bf2_ref[t]
        hj = _dotT(S, hprev)
        m = jnp.sum((hj * w * fcE).reshape(A, NN, F), axis=1)
        s2 = _dot(m, Wu_ref[t]) + bu_ref[t]
        g_s2 = g_h * _dsilu(s2)
        g_m = _dotBT(g_s2, Wu_ref[t])
        g_msg = jnp.broadcast_to(g_m[:, None, :], (A, NN, F)).reshape(E, F)
        g_hj = g_msg * w * fcE
        g_w = g_msg * hj * fcE
        g_fc = g_fc + jnp.sum(g_msg * hj * w, axis=1, keepdims=True)
        g_h = g_h + _dot(S, g_hj)
        g_s1 = _dotBT(g_w, Wf2_ref[t]) * _dsilu(s1)
        g_rbf = g_rbf + _dotBT(g_s1, Wf1_ref[t])
        return g_h, g_rbf, g_fc

    init = (g_h3, jnp.zeros((E, RES), jnp.float32),
            jnp.zeros((E, 1), jnp.float32))
    _, g_rbf, g_fc = jax.lax.fori_loop(0, T, step, init)
    return g_rbf, g_fc


def _body(R_ref, Z_ref, N_ref,
          cemb_ref, cWf1_ref, cbf1_ref, cWf2_ref, cbf2_ref, cWu_ref, cbu_ref,
          cWa_ref, cWp_ref,
          demb_ref, dWf1_ref, dbf1_ref, dWf2_ref, dbf2_ref, dWu_ref, dbu_ref,
          dWa_ref, dWp_ref,
          E_ref, F_ref, Q_ref, Bm_ref, D_ref,
          hsc_ref, hsd_ref):
    R = R_ref[0]
    Zr = Z_ref[0]
    Nf = N_ref[0].reshape(1, E)

    aio = jax.lax.broadcasted_iota(jnp.int32, (A, E), 0)
    S = (aio == Nf).astype(jnp.float32)

    Rj = _dotTH(S, R)
    Ri = jnp.broadcast_to(R[:, None, :], (A, NN, 3)).reshape(E, 3)
    diff = Ri - Rj
    Dd = jnp.sqrt(jnp.sum(diff * diff, axis=1, keepdims=True) + 1e-12)
    del diff, Rj, Ri

    centers = _centers()
    rbf = jnp.exp(-10.0 * (Dd - centers) ** 2)
    qq = jnp.clip(Dd / CUTOFF, 0.0, 1.0)
    fcE = 0.5 * (jnp.cos(jnp.pi * qq) + 1.0) * (Dd < CUTOFF).astype(jnp.float32)

    zio = jax.lax.broadcasted_iota(jnp.int32, (NZ, A), 0)
    ZohT = (zio == Zr).astype(jnp.float32)

    cWa = cWa_ref[...]; cWp = cWp_ref[...]
    dWa = dWa_ref[...]

    h0c = _dotT(ZohT, cemb_ref[...])
    h3c = _fwd(S, rbf, fcE, h0c, cWf1_ref, cbf1_ref, cWf2_ref, cbf2_ref,
               cWu_ref, cbu_ref, hsc_ref)
    Ai = _dotBT(h3c, cWa)

    h0d = _dotT(ZohT, demb_ref[...])
    _fwd(S, rbf, fcE, h0d, dWf1_ref, dbf1_ref, dWf2_ref, dbf2_ref,
         dWu_ref, dbu_ref, hsd_ref)
    Aid = _dotBT(hsd_ref[T], dWa)

    hjc = _dotT(S, h3c)
    hie = jnp.broadcast_to(h3c[:, None, :], (A, NN, F)).reshape(E, F)
    P = _dotBT(hie * hjc, cWp)

    Dinv = jnp.where(Dd > 1e-6, 1.0 / Dd, 0.0)
    Qi_e = jnp.broadcast_to(Ai[:, None, :], (A, NN, 1)).reshape(E, 1)
    Qj_e = _dotT(S, Ai)
    Ec = 0.5 * K * jnp.sum(Dinv * Qi_e * Qj_e)
    Eb = -0.25 * K * jnp.sum(Dinv * P * P)
    Etot = Ec + Eb + jnp.sum(Aid)

    gQe_i = (0.5 * K) * Dinv * Qj_e
    gQe_j = (0.5 * K) * Dinv * Qi_e
    gQ = jnp.sum(gQe_i.reshape(A, NN, 1), axis=1) + _dot(S, gQe_j)
    gP = (-0.5 * K) * Dinv * P
    gD = -((0.5 * K) * Qi_e * Qj_e - (0.25 * K) * P * P) * Dinv * Dinv

    g_h3c = _dot(gQ, cWa)
    gPhi = gP * hjc * cWp
    gPhj = gP * hie * cWp
    g_h3c = g_h3c + jnp.sum(gPhi.reshape(A, NN, F), axis=1) + _dot(S, gPhj)

    g_rbf_c, g_fc_c = _bwd(S, rbf, fcE, hsc_ref, g_h3c,
                           cWf1_ref, cbf1_ref, cWf2_ref, cbf2_ref,
                           cWu_ref, cbu_ref)

    g_h3d = jnp.broadcast_to(dWa, (A, F))
    g_rbf_d, g_fc_d = _bwd(S, rbf, fcE, hsd_ref, g_h3d,
                           dWf1_ref, dbf1_ref, dWf2_ref, dbf2_ref,
                           dWu_ref, dbu_ref)

    g_rbf = g_rbf_c + g_rbf_d
    g_fc = g_fc_c + g_fc_d

    gD = gD + jnp.sum(g_rbf * rbf * (-20.0) * (Dd - centers),
                      axis=1, keepdims=True)
    dfc = jnp.where(Dd < CUTOFF,
                    (-0.1 * np.pi) * jnp.sin((np.pi / CUTOFF) * Dd),
                    0.0)
    gD = gD + g_fc * dfc

    diff2 = (jnp.broadcast_to(R[:, None, :], (A, NN, 3)).reshape(E, 3)
             - _dotTH(S, R))
    gDu = (gD / Dd) * diff2
    Fb = -(jnp.sum(gDu.reshape(A, NN, 3), axis=1) - _dot(S, gDu))

    E_ref[...] = Etot.reshape(1, 1, 1)
    F_ref[0] = Fb
    Q_ref[0] = Ai
    Bm_ref[0] = P.reshape(A, NN)
    D_ref[0] = Dd.reshape(A, NN)


def kernel(R, Z, N, chg_embed, chg_Wf1, chg_bf1, chg_Wf2, chg_bf2, chg_Wu,
           chg_bu, chg_Wa, chg_Wp, dlt_embed, dlt_Wf1, dlt_bf1, dlt_Wf2,
           dlt_bf2, dlt_Wu, dlt_bu, dlt_Wa, dlt_Wp):
    cWa = chg_Wa.reshape(1, F)
    cWp = chg_Wp.reshape(1, F)
    dWa = dlt_Wa.reshape(1, F)
    dWp = dlt_Wp.reshape(1, F)

    full = lambda shape: pl.BlockSpec(shape, lambda b: (0,) * len(shape))
    in_specs = [
        pl.BlockSpec((1, A, 3), lambda b: (b, 0, 0)),
        pl.BlockSpec((1, 1, A), lambda b: (b, 0, 0)),
        pl.BlockSpec((1, A, NN), lambda b: (b, 0, 0)),
        full((NZ, F)), full((T, RES, F)), full((T, 1, F)), full((T, F, F)),
        full((T, 1, F)), full((T, F, F)), full((T, 1, F)), full((1, F)),
        full((1, F)),
        full((NZ, F)), full((T, RES, F)), full((T, 1, F)), full((T, F, F)),
        full((T, 1, F)), full((T, F, F)), full((T, 1, F)), full((1, F)),
        full((1, F)),
    ]
    out_specs = [
        pl.BlockSpec((1, 1, 1), lambda b: (b, 0, 0)),
        pl.BlockSpec((1, A, 3), lambda b: (b, 0, 0)),
        pl.BlockSpec((1, A, 1), lambda b: (b, 0, 0)),
        pl.BlockSpec((1, A, NN), lambda b: (b, 0, 0)),
        pl.BlockSpec((1, A, NN), lambda b: (b, 0, 0)),
    ]
    out_shape = [
        jax.ShapeDtypeStruct((B, 1, 1), jnp.float32),
        jax.ShapeDtypeStruct((B, A, 3), jnp.float32),
        jax.ShapeDtypeStruct((B, A, 1), jnp.float32),
        jax.ShapeDtypeStruct((B, A, NN), jnp.float32),
        jax.ShapeDtypeStruct((B, A, NN), jnp.float32),
    ]
    Ev, Fv, Qv, Bmv, Dv = pl.pallas_call(
        _body,
        grid=(B,),
        in_specs=in_specs,
        out_specs=out_specs,
        out_shape=out_shape,
        scratch_shapes=[
            pltpu.VMEM((T + 1, A, F), jnp.float32),
            pltpu.VMEM((T + 1, A, F), jnp.float32),
        ],
        compiler_params=pltpu.CompilerParams(
            vmem_limit_bytes=128 * 1024 * 1024),
    )(R, Z.astype(jnp.int32).reshape(B, 1, A), N.astype(jnp.int32),
      chg_embed, chg_Wf1, chg_bf1.reshape(T, 1, F), chg_Wf2,
      chg_bf2.reshape(T, 1, F), chg_Wu, chg_bu.reshape(T, 1, F),
      cWa, cWp,
      dlt_embed, dlt_Wf1, dlt_bf1.reshape(T, 1, F), dlt_Wf2,
      dlt_bf2.reshape(T, 1, F), dlt_Wu, dlt_bu.reshape(T, 1, F),
      dWa, dWp)
    return Ev.reshape(B, 1), Fv, Qv, Bmv, Dv
